# Initial kernel scaffold; baseline (speedup 1.0000x reference)
import jax
import jax.numpy as jnp
from jax.experimental import pallas as pl


def kernel(mu, x, edge_index, edge_w, W1, W2, W3, W4, W5, W7):
    raise NotImplementedError("write your pallas kernel here")


# speedup vs baseline: 75.0819x; 75.0819x over previous
import functools

import jax
import jax.numpy as jnp
from jax import lax
from jax.experimental import pallas as pl
from jax.experimental.pallas import tpu as pltpu
from jax.experimental.pallas import tpu_sc as plsc

T = 4
H = 128
NC = 2
NS = 16
L = 16
NW = NC * NS


def _sc_segsums(dst, ew, n_nodes):
    E = dst.shape[0]
    EPW = E // NW
    assert EPW * NW == E and EPW % L == 0
    NPAD = ((n_nodes + 1023) // 1024) * 1024
    NFLAT = 3 * NPAD
    STRIPE = NFLAT // NS
    assert STRIPE % L == 0

    mesh = plsc.VectorSubcoreMesh(core_axis_name="c", subcore_axis_name="s")

    @functools.partial(
        pl.kernel,
        out_type=jax.ShapeDtypeStruct((NC, NFLAT), jnp.float32),
        mesh=mesh,
        compiler_params=pltpu.CompilerParams(needs_layout_passes=False),
        scratch_types=[
            pltpu.VMEM((EPW,), jnp.int32),
            pltpu.VMEM((EPW,), jnp.float32),
            pltpu.VMEM((NFLAT,), jnp.float32),
            pltpu.VMEM((NS, STRIPE), jnp.float32),
            pltpu.VMEM((STRIPE,), jnp.float32),
            pltpu.VMEM_SHARED((NS, NFLAT), jnp.float32),
        ],
    )
    def sc_seg(dst_hbm, ew_hbm, out_hbm, idx_v, w_v, acc_v, red_v, res_v, shared):
        c = lax.axis_index("c")
        s = lax.axis_index("s")
        wid = s * NC + c
        base = wid * EPW
        pltpu.sync_copy(dst_hbm.at[pl.ds(base, EPW)], idx_v)
        pltpu.sync_copy(ew_hbm.at[pl.ds(base, EPW)], w_v)

        def zbody(i, carry):
            acc_v[pl.ds(i * L, L)] = jnp.zeros((L,), jnp.float32)
            return carry
        lax.fori_loop(0, NFLAT // L, zbody, 0)

        ones = jnp.ones((L,), jnp.float32)

        def ebody(i, carry):
            idx = idx_v[pl.ds(i * L, L)]
            w = w_v[pl.ds(i * L, L)]
            i3 = idx * 3
            plsc.addupdate_scatter(acc_v, [i3], ones)
            plsc.addupdate_scatter(acc_v, [i3 + 1], jnp.maximum(w, 0.0))
            plsc.addupdate_scatter(acc_v, [i3 + 2], jnp.maximum(-w, 0.0))
            return carry
        lax.fori_loop(0, EPW // L, ebody, 0)

        pltpu.sync_copy(acc_v, shared.at[s])
        plsc.subcore_barrier()
        pltpu.sync_copy(shared.at[:, pl.ds(s * STRIPE, STRIPE)], red_v)

        def rbody(j, carry):
            tot = red_v[0, pl.ds(j * L, L)]
            for k in range(1, NS):
                tot = tot + red_v[k, pl.ds(j * L, L)]
            res_v[pl.ds(j * L, L)] = tot
            return carry
        lax.fori_loop(0, STRIPE // L, rbody, 0)
        pltpu.sync_copy(res_v, out_hbm.at[c, pl.ds(s * STRIPE, STRIPE)])

    part = sc_seg(dst, ew)
    return part.reshape(NC, NPAD, 3)


def _tc_body(part_ref, mu_ref, x_ref, W1r, W2r, W3r, W4r, W5r, W7r, y_ref, pool_ref):
    i = pl.program_id(0)
    red = part_ref[0] + part_ref[1]
    deg = red[:, 0:1]
    sp = red[:, 1:2]
    sm = red[:, 2:3]
    mu = mu_ref[...]
    xb = x_ref[...]
    for t in range(T):
        w4 = W4r[t]
        cpt = jnp.dot(jnp.maximum(w4, 0.0), W3r[t], preferred_element_type=jnp.float32)
        cmt = jnp.dot(jnp.maximum(-w4, 0.0), W3r[t], preferred_element_type=jnp.float32)
        b = xb * W1r[t] + sp * cpt + sm * cmt
        mu = jnp.maximum(
            jnp.dot(deg * mu, W2r[t], preferred_element_type=jnp.float32) + b, 0.0)
    nv = jnp.dot(mu, W7r[...], preferred_element_type=jnp.float32)
    y_ref[...] = jnp.dot(jnp.maximum(nv, 0.0), W5r[H:, :],
                         preferred_element_type=jnp.float32)
    psum = jnp.sum(mu, axis=0, keepdims=True)

    @pl.when(i == 0)
    def _():
        pool_ref[...] = psum

    @pl.when(i > 0)
    def _():
        pool_ref[...] += psum


def _finish_body(y_ref, pool_ref, W5r, out_ref):
    const = jnp.dot(jnp.maximum(pool_ref[...], 0.0), W5r[:H, :],
                    preferred_element_type=jnp.float32)
    out_ref[...] = y_ref[...] + const


def kernel(mu, x, edge_index, edge_w, W1, W2, W3, W4, W5, W7):
    N = mu.shape[0]
    dst = edge_index[1]
    ew = edge_w[:, 0]
    part = _sc_segsums(dst, ew, N)

    R = 1000
    NB = N // R
    assert NB * R == N

    y, pool = pl.pallas_call(
        _tc_body,
        grid=(NB,),
        in_specs=[
            pl.BlockSpec((NC, R, 3), lambda i: (0, i, 0)),
            pl.BlockSpec((R, H), lambda i: (i, 0)),
            pl.BlockSpec((R, 1), lambda i: (i, 0)),
            pl.BlockSpec((T, 1, H), lambda i: (0, 0, 0)),
            pl.BlockSpec((T, H, H), lambda i: (0, 0, 0)),
            pl.BlockSpec((T, H, H), lambda i: (0, 0, 0)),
            pl.BlockSpec((T, 1, H), lambda i: (0, 0, 0)),
            pl.BlockSpec((2 * H, 1), lambda i: (0, 0)),
            pl.BlockSpec((H, H), lambda i: (0, 0)),
        ],
        out_specs=[
            pl.BlockSpec((R, 1), lambda i: (i, 0)),
            pl.BlockSpec((1, H), lambda i: (0, 0)),
        ],
        out_shape=[
            jax.ShapeDtypeStruct((N, 1), jnp.float32),
            jax.ShapeDtypeStruct((1, H), jnp.float32),
        ],
    )(part, mu, x, W1, W2, W3, W4, W5, W7)

    out = pl.pallas_call(
        _finish_body,
        grid=(NB,),
        in_specs=[
            pl.BlockSpec((R, 1), lambda i: (i, 0)),
            pl.BlockSpec((1, H), lambda i: (0, 0)),
            pl.BlockSpec((2 * H, 1), lambda i: (0, 0)),
        ],
        out_specs=pl.BlockSpec((R, 1), lambda i: (i, 0)),
        out_shape=jax.ShapeDtypeStruct((N, 1), jnp.float32),
    )(y, pool, W5)
    return out[:, 0]
